# baseline (speedup 1.0000x reference)
import dataclasses
import functools

import jax
import jax.numpy as jnp
from jax import lax
from jax.experimental import pallas as pl
from jax.experimental.pallas import tpu as pltpu
from jax.experimental.pallas import tpu_sc as plsc

B = 16384
D = 64
NC = 2
NS = 16
L = 16
NW = NC * NS
BPW = B // NW
CHUNK = 128
NCH = BPW // CHUNK
GPC = CHUNK // L

_mesh = plsc.VectorSubcoreMesh(core_axis_name="c", subcore_axis_name="s")

_cp = pltpu.CompilerParams()
if "needs_layout_passes" in pltpu.CompilerParams.__dataclass_fields__:
    _cp = dataclasses.replace(_cp, needs_layout_passes=False)
if "use_tc_tiling_on_sc" in pltpu.CompilerParams.__dataclass_fields__:
    _cp = dataclasses.replace(_cp, use_tc_tiling_on_sc=False)


@functools.partial(
    pl.kernel,
    mesh=_mesh,
    compiler_params=_cp,
    out_type=jax.ShapeDtypeStruct((B,), jnp.float32),
    scratch_types=[
        pltpu.VMEM((NCH, CHUNK), jnp.int32),
        pltpu.VMEM((NCH, CHUNK), jnp.int32),
        pltpu.VMEM((CHUNK, D), jnp.float32),
        pltpu.VMEM((CHUNK, D), jnp.float32),
        pltpu.VMEM((BPW,), jnp.float32),
        pltpu.SemaphoreType.DMA,
    ],
)
def _mf_dot_kernel(uid_hbm, iid_hbm, utab_hbm, itab_hbm, out_hbm,
                   uidx, iidx, urows, irows, outv, sem):
    wid = lax.axis_index("s") * NC + lax.axis_index("c")
    pltpu.sync_copy(uid_hbm.at[wid], uidx)
    pltpu.sync_copy(iid_hbm.at[wid], iidx)

    @pl.loop(0, NCH)
    def _chunk(c):
        ucp = pltpu.async_copy(utab_hbm.at[uidx.at[c]], urows, sem)
        icp = pltpu.async_copy(itab_hbm.at[iidx.at[c]], irows, sem)
        ucp.wait()
        icp.wait()

        @pl.loop(0, GPC)
        def _group(g):
            rows = g * L + lax.iota(jnp.int32, L)
            acc = jnp.zeros((L,), jnp.float32)
            for d in range(D):
                cols = jnp.full((L,), d, jnp.int32)
                u = plsc.load_gather(urows, [rows, cols])
                v = plsc.load_gather(irows, [rows, cols])
                acc = acc + u * v
            outv[pl.ds(c * CHUNK + g * L, L)] = acc

    pltpu.sync_copy(outv, out_hbm.at[pl.ds(wid * BPW, BPW)])


def kernel(user_ids, item_ids, user_table, item_table):
    uid = user_ids.reshape(NW, NCH, CHUNK)
    iid = item_ids.reshape(NW, NCH, CHUNK)
    out = _mf_dot_kernel(uid, iid, user_table, item_table)
    return out.reshape(B, 1)
